# baseline (speedup 1.0000x reference)
import functools

import jax
import jax.numpy as jnp
import numpy as np
from jax import lax
from jax.experimental import pallas as pl
from jax.experimental.pallas import tpu as pltpu
from jax.experimental.pallas import tpu_sc as plsc

_N = 10000
_E = 320000
_D = 128
_K = 3
_PA = 1.0
_PB = 1.0
_LO = -1.0
_HI = 1.0

_NC = 2
_NS = 16
_NW = _NC * _NS

_CH = 128
_EPT = 10240
_NCHT = _EPT // _CH
_NQ = 5
_QCH = _NCHT // _NQ
_EP = _NW * _EPT
_NP = 10240
_RPT = _NP // _NS


@functools.lru_cache(maxsize=None)
def _build_sc_kernels():
    mesh = plsc.VectorSubcoreMesh(
        core_axis_name="c", subcore_axis_name="s", num_cores=_NC, num_subcores=_NS
    )

    n_full = _E // (_CH * 79)
    tail_ch = (_E - n_full * 79 * _CH) // _CH

    @functools.partial(
        pl.kernel,
        out_type=jax.ShapeDtypeStruct((_NC, _NP), jnp.float32),
        mesh=mesh,
        scratch_types=[
            pltpu.VMEM((80 * _CH,), jnp.int32),
            pltpu.VMEM((_CH,), jnp.float32),
            pltpu.VMEM_SHARED((_NP,), jnp.float32),
            pltpu.SemaphoreType.DMA,
        ],
    )
    def deg_kernel(ei_hbm, zeros1_hbm, ones1_hbm, degp_hbm, row_v, ones_v, acc, sem_s):
        c = lax.axis_index("c")
        s = lax.axis_index("s")
        wid = c * _NS + s
        nch = jnp.where(wid < n_full, 79, tail_ch)
        base = wid * (79 * _CH)
        pltpu.sync_copy(zeros1_hbm, acc.at[pl.ds(s * _RPT, _RPT)])
        pltpu.sync_copy(ones1_hbm, ones_v)

        @pl.when(wid < n_full)
        def _load_full():
            pltpu.sync_copy(ei_hbm.at[0, pl.ds(base, 79 * _CH)], row_v.at[pl.ds(0, 79 * _CH)])

        @pl.when(wid >= n_full)
        def _load_tail():
            pltpu.sync_copy(
                ei_hbm.at[0, pl.ds(base, tail_ch * _CH)], row_v.at[pl.ds(0, tail_ch * _CH)]
            )

        plsc.subcore_barrier()

        def body(j, carry):
            pltpu.async_copy(ones_v, acc.at[row_v.at[pl.ds(j * _CH, _CH)]], sem_s, add=True)
            return carry

        lax.fori_loop(0, nch, body, 0)

        def drain(j, carry):
            pltpu.make_async_copy(
                ones_v, acc.at[row_v.at[pl.ds(j * _CH, _CH)]], sem_s
            ).wait()
            return carry

        lax.fori_loop(0, nch, drain, 0)
        plsc.subcore_barrier()
        pltpu.sync_copy(acc.at[pl.ds(s * _RPT, _RPT)], degp_hbm.at[c].at[pl.ds(s * _RPT, _RPT)])

    @functools.partial(
        pl.kernel,
        out_type=[
            jax.ShapeDtypeStruct((_NP, _D), jnp.float32),
            jax.ShapeDtypeStruct((_NP, _D), jnp.float32),
        ],
        mesh=mesh,
        scratch_types=[
            pltpu.VMEM((2, _QCH * _CH), jnp.int32),
            pltpu.VMEM((2, _QCH * _CH), jnp.int32),
            pltpu.VMEM((2, _CH, _D), jnp.float32),
            pltpu.VMEM_SHARED((_NP, _D), jnp.float32),
            pltpu.SemaphoreType.DMA,
            pltpu.SemaphoreType.DMA,
        ],
    )
    def spmm_kernel(
        y_hbm, colp_hbm, rowp_hbm, zeros2_hbm, part0_hbm, part1_hbm,
        col_v, row_v, buf, acc, sem, sem_i,
    ):
        c = lax.axis_index("c")
        s = lax.axis_index("s")
        wid = c * _NS + s
        pltpu.sync_copy(zeros2_hbm, acc.at[pl.ds(s * _RPT, _RPT)])
        ebase = wid * _EPT
        qe = _QCH * _CH
        pltpu.sync_copy(colp_hbm.at[pl.ds(ebase, qe)], col_v.at[0])
        pltpu.sync_copy(rowp_hbm.at[pl.ds(ebase, qe)], row_v.at[0])
        pltpu.async_copy(colp_hbm.at[pl.ds(ebase + qe, qe)], col_v.at[1], sem_i)
        pltpu.async_copy(rowp_hbm.at[pl.ds(ebase + qe, qe)], row_v.at[1], sem_i)
        plsc.subcore_barrier()

        pltpu.async_copy(y_hbm.at[col_v.at[0, pl.ds(0, _CH)]], buf.at[0], sem)

        def body(j, carry):
            q = j // _QCH
            jj = j - q * _QCH
            slot = lax.rem(q, 2)
            cur = lax.rem(j, 2)

            @pl.when(jnp.logical_and(jj == 0, jnp.logical_and(q >= 1, q < _NQ - 1)))
            def _prefetch_idx():
                nslot = 1 - slot
                off = ebase + (q + 1) * qe
                pltpu.async_copy(colp_hbm.at[pl.ds(off, qe)], col_v.at[nslot], sem_i)
                pltpu.async_copy(rowp_hbm.at[pl.ds(off, qe)], row_v.at[nslot], sem_i)

            pltpu.make_async_copy(
                y_hbm.at[col_v.at[slot, pl.ds(jj * _CH, _CH)]], buf.at[cur], sem
            ).wait()

            @pl.when(j < _NCHT - 1)
            def _start_next():
                nj = j + 1
                nq = nj // _QCH
                njj = nj - nq * _QCH
                nslot = lax.rem(nq, 2)

                @pl.when(njj == 0)
                def _wait_idx():
                    off = ebase + nq * qe
                    pltpu.make_async_copy(
                        colp_hbm.at[pl.ds(off, qe)], col_v.at[nslot], sem_i
                    ).wait()
                    pltpu.make_async_copy(
                        rowp_hbm.at[pl.ds(off, qe)], row_v.at[nslot], sem_i
                    ).wait()

                pltpu.async_copy(
                    y_hbm.at[col_v.at[nslot, pl.ds(njj * _CH, _CH)]], buf.at[1 - cur], sem
                )

            pltpu.sync_copy(
                buf.at[cur], acc.at[row_v.at[slot, pl.ds(jj * _CH, _CH)]], add=True
            )
            return carry

        lax.fori_loop(0, _NCHT, body, 0)
        plsc.subcore_barrier()

        @pl.when(c == 0)
        def _drain0():
            pltpu.sync_copy(acc.at[pl.ds(s * _RPT, _RPT)], part0_hbm.at[pl.ds(s * _RPT, _RPT)])

        @pl.when(c == 1)
        def _drain1():
            pltpu.sync_copy(acc.at[pl.ds(s * _RPT, _RPT)], part1_hbm.at[pl.ds(s * _RPT, _RPT)])

    return deg_kernel, spmm_kernel


_R = 2000


def _disy_body(dt_ref, x_ref, y_ref, dis_ref):
    deg = lax.dot_general(
        dt_ref[...],
        jnp.ones((_NC, 1), jnp.float32),
        (((0,), (0,)), ((), ())),
        preferred_element_type=jnp.float32,
    )[:_N]
    deg = jnp.where(deg < 0.5, deg + 1.0, deg)
    dis = lax.rsqrt(deg)
    dis_ref[...] = dis
    y_ref[...] = dis * x_ref[...]


def _combine_first_body(p0_ref, p1_ref, dis_ref, xc_ref, cf_ref, xn_ref, yn_ref):
    seg = p0_ref[...] + p1_ref[...]
    dis = dis_ref[...]
    xn = cf_ref[0, 0] * (dis * seg) + cf_ref[0, 1] * xc_ref[...]
    xn_ref[...] = xn
    yn_ref[...] = dis * xn


def _combine_mid_body(p0_ref, p1_ref, dis_ref, xc_ref, xp_ref, cf_ref, xn_ref, yn_ref):
    seg = p0_ref[...] + p1_ref[...]
    dis = dis_ref[...]
    xn = (
        cf_ref[0, 0] * (dis * seg)
        + cf_ref[0, 1] * xc_ref[...]
        + cf_ref[0, 2] * xp_ref[...]
    )
    xn_ref[...] = xn
    yn_ref[...] = dis * xn


def _combine_mm_body(
    p0_ref, p1_ref, dis_ref, x2_ref, x1_ref, cf_ref, x0_ref, w_ref, b_ref, o_ref
):
    seg = p0_ref[...] + p1_ref[...]
    x3 = (
        cf_ref[0, 0] * (dis_ref[...] * seg)
        + cf_ref[0, 1] * x2_ref[...]
        + cf_ref[0, 2] * x1_ref[...]
    )
    acc = b_ref[...] + jnp.dot(x0_ref[...], w_ref[0], preferred_element_type=jnp.float32)
    acc = acc + jnp.dot(x1_ref[...], w_ref[1], preferred_element_type=jnp.float32)
    acc = acc + jnp.dot(x2_ref[...], w_ref[2], preferred_element_type=jnp.float32)
    acc = acc + jnp.dot(x3, w_ref[3], preferred_element_type=jnp.float32)
    o_ref[...] = acc


_EBLKS = 10
_EBLK = _EP // _EBLKS


def _repack_body(e_ref, rowp_ref, colp_ref):
    i = pl.program_id(0)
    eg = i * _EBLK + lax.broadcasted_iota(jnp.int32, (_EBLK,), 0)
    k = eg - _E
    in_pad = eg >= _E
    pr = _N + (k - (k // (_NP - _N)) * (_NP - _N))
    pc = k - (k // _N) * _N
    rowp_ref[...] = jnp.where(in_pad, pr, e_ref[0, :])
    colp_ref[...] = jnp.where(in_pad, pc, e_ref[1, :])


def _row_spec(width):
    return pl.BlockSpec((_R, width), lambda i: (i, 0))


def _rep_spec(shape):
    ndim = len(shape)
    return pl.BlockSpec(shape, lambda i, _nd=ndim: (0,) * _nd)


_GRID = (_N // _R,)

_disy_call = pl.pallas_call(
    _disy_body,
    grid=(1,),
    in_specs=[
        pl.BlockSpec((_NC, _NP), lambda i: (0, 0)),
        pl.BlockSpec((_N, _D), lambda i: (0, 0)),
    ],
    out_specs=[
        pl.BlockSpec((_N, _D), lambda i: (0, 0)),
        pl.BlockSpec((_N, 1), lambda i: (0, 0)),
    ],
    out_shape=[
        jax.ShapeDtypeStruct((_N, _D), jnp.float32),
        jax.ShapeDtypeStruct((_N, 1), jnp.float32),
    ],
)

_combine_xy_shape = [
    jax.ShapeDtypeStruct((_N, _D), jnp.float32),
    jax.ShapeDtypeStruct((_N, _D), jnp.float32),
]

_combine_first_call = pl.pallas_call(
    _combine_first_body,
    grid=_GRID,
    in_specs=[_row_spec(_D), _row_spec(_D), _row_spec(1), _row_spec(_D), _rep_spec((1, 3))],
    out_specs=[_row_spec(_D), _row_spec(_D)],
    out_shape=_combine_xy_shape,
)

_combine_mid_call = pl.pallas_call(
    _combine_mid_body,
    grid=_GRID,
    in_specs=[
        _row_spec(_D), _row_spec(_D), _row_spec(1), _row_spec(_D), _row_spec(_D),
        _rep_spec((1, 3)),
    ],
    out_specs=[_row_spec(_D), _row_spec(_D)],
    out_shape=_combine_xy_shape,
)

_combine_mm_call = pl.pallas_call(
    _combine_mm_body,
    grid=_GRID,
    in_specs=[
        _row_spec(_D), _row_spec(_D), _row_spec(1), _row_spec(_D), _row_spec(_D),
        _rep_spec((1, 3)),
        _row_spec(_D),
        _rep_spec((_K + 1, _D, _D)),
        _rep_spec((1, _D)),
    ],
    out_specs=_row_spec(_D),
    out_shape=jax.ShapeDtypeStruct((_N, _D), jnp.float32),
)

_repack_call = pl.pallas_call(
    _repack_body,
    grid=(_EBLKS,),
    in_specs=[pl.BlockSpec((2, _EBLK), lambda i: (0, i))],
    out_specs=[
        pl.BlockSpec((_EBLK,), lambda i: (i,)),
        pl.BlockSpec((_EBLK,), lambda i: (i,)),
    ],
    out_shape=[
        jax.ShapeDtypeStruct((_EP,), jnp.int32),
        jax.ShapeDtypeStruct((_EP,), jnp.int32),
    ],
)


def _coef_schedule(alphas):
    coef1 = (_PA - _PB) / 2 - (_PA + _PB + 2) / 2 * ((_LO + _HI) / (_HI - _LO))
    coef2 = (_PA + _PB + 2) / (_HI - _LO)
    scheds = [(alphas[0] * coef2, alphas[0] * coef1, alphas[0] * 0.0)]
    for L in range(2, _K + 1):
        coef_l = 2 * L * (L + _PA + _PB) * (2 * L - 2 + _PA + _PB)
        coef_lm1_1 = (2 * L + _PA + _PB - 1) * (2 * L + _PA + _PB) * (2 * L + _PA + _PB - 2)
        coef_lm1_2 = (2 * L + _PA + _PB - 1) * (_PA**2 - _PB**2)
        coef_lm2 = 2 * (L - 1 + _PA) * (L - 1 + _PB) * (2 * L + _PA + _PB)
        tmp1 = alphas[L - 1] * (coef_lm1_1 / coef_l)
        tmp2 = alphas[L - 1] * (coef_lm1_2 / coef_l)
        tmp3 = alphas[L - 1] * alphas[L - 2] * (coef_lm2 / coef_l)
        tmp1_2 = tmp1 * (2.0 / (_HI - _LO))
        tmp2_2 = tmp1 * ((_HI + _LO) / (_HI - _LO)) + tmp2
        scheds.append((tmp1_2, -tmp2_2, -tmp3))
    return scheds


def kernel(x, edge_index, alphas_param, W, bias):
    rowp, colp = _repack_call(edge_index)
    zeros1 = jnp.zeros((_RPT,), jnp.float32)
    ones1 = jnp.ones((_CH,), jnp.float32)
    zeros2 = jnp.zeros((_RPT, _D), jnp.float32)

    deg_kernel, spmm_kernel = _build_sc_kernels()
    degp = deg_kernel(edge_index, zeros1, ones1)
    y, dis = _disy_call(degp, x)

    alphas = jnp.tanh(alphas_param)
    scheds = _coef_schedule(alphas)
    cfs = [jnp.stack(s).reshape(1, 3).astype(jnp.float32) for s in scheds]

    p0, p1 = spmm_kernel(y, colp, rowp, zeros2)
    x1, y = _combine_first_call(p0, p1, dis, x, cfs[0])
    p0, p1 = spmm_kernel(y, colp, rowp, zeros2)
    x2, y = _combine_mid_call(p0, p1, dis, x1, x, cfs[1])
    p0, p1 = spmm_kernel(y, colp, rowp, zeros2)

    wt = W.T.reshape(_K + 1, _D, _D)
    return _combine_mm_call(p0, p1, dis, x2, x1, cfs[2], x, wt, bias.reshape(1, _D))


# speedup vs baseline: 1.0290x; 1.0290x over previous
import functools

import jax
import jax.numpy as jnp
import numpy as np
from jax import lax
from jax.experimental import pallas as pl
from jax.experimental.pallas import tpu as pltpu
from jax.experimental.pallas import tpu_sc as plsc

_N = 10000
_E = 320000
_D = 128
_K = 3
_PA = 1.0
_PB = 1.0
_LO = -1.0
_HI = 1.0

_NC = 2
_NS = 16
_NW = _NC * _NS

_CH = 128
_EPT = 10240
_NCHT = _EPT // _CH
_NQ = 5
_QCH = _NCHT // _NQ
_EP = _NW * _EPT
_NP = 10240
_RPT = _NP // _NS


@functools.lru_cache(maxsize=None)
def _build_sc_kernels():
    mesh = plsc.VectorSubcoreMesh(
        core_axis_name="c", subcore_axis_name="s", num_cores=_NC, num_subcores=_NS
    )

    n_full = _E // (_CH * 79)
    tail_ch = (_E - n_full * 79 * _CH) // _CH

    @functools.partial(
        pl.kernel,
        out_type=jax.ShapeDtypeStruct((_NC, _NP), jnp.float32),
        mesh=mesh,
        scratch_types=[
            pltpu.VMEM((80 * _CH,), jnp.int32),
            pltpu.VMEM((_CH,), jnp.float32),
            pltpu.VMEM_SHARED((_NP,), jnp.float32),
            pltpu.SemaphoreType.DMA,
        ],
    )
    def deg_kernel(ei_hbm, zeros1_hbm, ones1_hbm, degp_hbm, row_v, ones_v, acc, sem_s):
        c = lax.axis_index("c")
        s = lax.axis_index("s")
        wid = c * _NS + s
        nch = jnp.where(wid < n_full, 79, tail_ch)
        base = wid * (79 * _CH)
        pltpu.sync_copy(zeros1_hbm, acc.at[pl.ds(s * _RPT, _RPT)])
        pltpu.sync_copy(ones1_hbm, ones_v)

        @pl.when(wid < n_full)
        def _load_full():
            pltpu.sync_copy(ei_hbm.at[0, pl.ds(base, 79 * _CH)], row_v.at[pl.ds(0, 79 * _CH)])

        @pl.when(wid >= n_full)
        def _load_tail():
            pltpu.sync_copy(
                ei_hbm.at[0, pl.ds(base, tail_ch * _CH)], row_v.at[pl.ds(0, tail_ch * _CH)]
            )

        plsc.subcore_barrier()

        def body(j, carry):
            pltpu.async_copy(ones_v, acc.at[row_v.at[pl.ds(j * _CH, _CH)]], sem_s, add=True)
            return carry

        lax.fori_loop(0, nch, body, 0)

        def drain(j, carry):
            pltpu.make_async_copy(
                ones_v, acc.at[row_v.at[pl.ds(j * _CH, _CH)]], sem_s
            ).wait()
            return carry

        lax.fori_loop(0, nch, drain, 0)
        plsc.subcore_barrier()
        pltpu.sync_copy(acc.at[pl.ds(s * _RPT, _RPT)], degp_hbm.at[c].at[pl.ds(s * _RPT, _RPT)])

    @functools.partial(
        pl.kernel,
        out_type=[
            jax.ShapeDtypeStruct((_NP, _D), jnp.float32),
            jax.ShapeDtypeStruct((_NP, _D), jnp.float32),
        ],
        mesh=mesh,
        scratch_types=[
            pltpu.VMEM((2, _QCH, _CH), jnp.int32),
            pltpu.VMEM((2, _QCH, _CH), jnp.int32),
            pltpu.VMEM((2, _CH, _D), jnp.float32),
            pltpu.VMEM_SHARED((_NP, _D), jnp.float32),
            pltpu.SemaphoreType.DMA,
            pltpu.SemaphoreType.DMA,
        ],
    )
    def spmm_kernel(
        y_hbm, colp_hbm, rowp_hbm, zeros2_hbm, part0_hbm, part1_hbm,
        col_v, row_v, buf, acc, sem, sem_i,
    ):
        c = lax.axis_index("c")
        s = lax.axis_index("s")
        wid = c * _NS + s
        pltpu.sync_copy(zeros2_hbm, acc.at[pl.ds(s * _RPT, _RPT)])
        pltpu.sync_copy(colp_hbm.at[wid, pl.ds(0, _QCH)], col_v.at[0])
        pltpu.sync_copy(rowp_hbm.at[wid, pl.ds(0, _QCH)], row_v.at[0])
        pltpu.async_copy(colp_hbm.at[wid, pl.ds(_QCH, _QCH)], col_v.at[1], sem_i)
        pltpu.async_copy(rowp_hbm.at[wid, pl.ds(_QCH, _QCH)], row_v.at[1], sem_i)
        plsc.subcore_barrier()

        pltpu.async_copy(y_hbm.at[col_v.at[0, 0]], buf.at[0], sem)

        def body(j, carry):
            q = j // _QCH
            jj = j - q * _QCH
            slot = lax.rem(q, 2)
            cur = lax.rem(j, 2)

            @pl.when(jnp.logical_and(jj == 0, jnp.logical_and(q >= 1, q < _NQ - 1)))
            def _prefetch_idx():
                nslot = 1 - slot
                off = (q + 1) * _QCH
                pltpu.async_copy(colp_hbm.at[wid, pl.ds(off, _QCH)], col_v.at[nslot], sem_i)
                pltpu.async_copy(rowp_hbm.at[wid, pl.ds(off, _QCH)], row_v.at[nslot], sem_i)

            pltpu.make_async_copy(y_hbm.at[col_v.at[slot, jj]], buf.at[cur], sem).wait()

            @pl.when(j < _NCHT - 1)
            def _start_next():
                nj = j + 1
                nq = nj // _QCH
                njj = nj - nq * _QCH
                nslot = lax.rem(nq, 2)

                @pl.when(njj == 0)
                def _wait_idx():
                    off = nq * _QCH
                    pltpu.make_async_copy(
                        colp_hbm.at[wid, pl.ds(off, _QCH)], col_v.at[nslot], sem_i
                    ).wait()
                    pltpu.make_async_copy(
                        rowp_hbm.at[wid, pl.ds(off, _QCH)], row_v.at[nslot], sem_i
                    ).wait()

                pltpu.async_copy(y_hbm.at[col_v.at[nslot, njj]], buf.at[1 - cur], sem)

            pltpu.sync_copy(buf.at[cur], acc.at[row_v.at[slot, jj]], add=True)
            return carry

        lax.fori_loop(0, _NCHT, body, 0)
        plsc.subcore_barrier()

        @pl.when(c == 0)
        def _drain0():
            pltpu.sync_copy(acc.at[pl.ds(s * _RPT, _RPT)], part0_hbm.at[pl.ds(s * _RPT, _RPT)])

        @pl.when(c == 1)
        def _drain1():
            pltpu.sync_copy(acc.at[pl.ds(s * _RPT, _RPT)], part1_hbm.at[pl.ds(s * _RPT, _RPT)])

    return deg_kernel, spmm_kernel


_R = 2000


def _disy_body(dt_ref, x_ref, y_ref, dis_ref):
    deg = lax.dot_general(
        dt_ref[...],
        jnp.ones((_NC, 1), jnp.float32),
        (((0,), (0,)), ((), ())),
        preferred_element_type=jnp.float32,
    )[:_N]
    deg = jnp.where(deg < 0.5, deg + 1.0, deg)
    dis = lax.rsqrt(deg)
    dis_ref[...] = dis
    y_ref[...] = dis * x_ref[...]


def _combine_first_body(p0_ref, p1_ref, dis_ref, xc_ref, cf_ref, xn_ref, yn_ref):
    seg = p0_ref[...] + p1_ref[...]
    dis = dis_ref[...]
    xn = cf_ref[0, 0] * (dis * seg) + cf_ref[0, 1] * xc_ref[...]
    xn_ref[...] = xn
    yn_ref[...] = dis * xn


def _combine_mid_body(p0_ref, p1_ref, dis_ref, xc_ref, xp_ref, cf_ref, xn_ref, yn_ref):
    seg = p0_ref[...] + p1_ref[...]
    dis = dis_ref[...]
    xn = (
        cf_ref[0, 0] * (dis * seg)
        + cf_ref[0, 1] * xc_ref[...]
        + cf_ref[0, 2] * xp_ref[...]
    )
    xn_ref[...] = xn
    yn_ref[...] = dis * xn


def _combine_mm_body(
    p0_ref, p1_ref, dis_ref, x2_ref, x1_ref, cf_ref, x0_ref, w_ref, b_ref, o_ref
):
    seg = p0_ref[...] + p1_ref[...]
    x3 = (
        cf_ref[0, 0] * (dis_ref[...] * seg)
        + cf_ref[0, 1] * x2_ref[...]
        + cf_ref[0, 2] * x1_ref[...]
    )
    acc = b_ref[...] + jnp.dot(x0_ref[...], w_ref[0], preferred_element_type=jnp.float32)
    acc = acc + jnp.dot(x1_ref[...], w_ref[1], preferred_element_type=jnp.float32)
    acc = acc + jnp.dot(x2_ref[...], w_ref[2], preferred_element_type=jnp.float32)
    acc = acc + jnp.dot(x3, w_ref[3], preferred_element_type=jnp.float32)
    o_ref[...] = acc


_EBLKS = 10
_EBLK = _EP // _EBLKS


_EROWS = 8
_ECOLS = _EBLK // _EROWS


def _repack_body(e_ref, rowp_ref, colp_ref):
    i = pl.program_id(0)
    eg = (
        i * _EBLK
        + lax.broadcasted_iota(jnp.int32, (_EROWS, _ECOLS), 0) * _ECOLS
        + lax.broadcasted_iota(jnp.int32, (_EROWS, _ECOLS), 1)
    )
    k = eg - _E
    in_pad = eg >= _E
    pr = _N + (k - (k // (_NP - _N)) * (_NP - _N))
    pc = k - (k // _N) * _N
    rowp_ref[...] = jnp.where(in_pad, pr, e_ref[0:1, :].reshape(_EROWS, _ECOLS))
    colp_ref[...] = jnp.where(in_pad, pc, e_ref[1:2, :].reshape(_EROWS, _ECOLS))


def _row_spec(width):
    return pl.BlockSpec((_R, width), lambda i: (i, 0))


def _rep_spec(shape):
    ndim = len(shape)
    return pl.BlockSpec(shape, lambda i, _nd=ndim: (0,) * _nd)


_GRID = (_N // _R,)

_disy_call = pl.pallas_call(
    _disy_body,
    grid=(1,),
    in_specs=[
        pl.BlockSpec((_NC, _NP), lambda i: (0, 0)),
        pl.BlockSpec((_N, _D), lambda i: (0, 0)),
    ],
    out_specs=[
        pl.BlockSpec((_N, _D), lambda i: (0, 0)),
        pl.BlockSpec((_N, 1), lambda i: (0, 0)),
    ],
    out_shape=[
        jax.ShapeDtypeStruct((_N, _D), jnp.float32),
        jax.ShapeDtypeStruct((_N, 1), jnp.float32),
    ],
)

_combine_xy_shape = [
    jax.ShapeDtypeStruct((_N, _D), jnp.float32),
    jax.ShapeDtypeStruct((_N, _D), jnp.float32),
]

_combine_first_call = pl.pallas_call(
    _combine_first_body,
    grid=_GRID,
    in_specs=[_row_spec(_D), _row_spec(_D), _row_spec(1), _row_spec(_D), _rep_spec((1, 3))],
    out_specs=[_row_spec(_D), _row_spec(_D)],
    out_shape=_combine_xy_shape,
)

_combine_mid_call = pl.pallas_call(
    _combine_mid_body,
    grid=_GRID,
    in_specs=[
        _row_spec(_D), _row_spec(_D), _row_spec(1), _row_spec(_D), _row_spec(_D),
        _rep_spec((1, 3)),
    ],
    out_specs=[_row_spec(_D), _row_spec(_D)],
    out_shape=_combine_xy_shape,
)

_combine_mm_call = pl.pallas_call(
    _combine_mm_body,
    grid=_GRID,
    in_specs=[
        _row_spec(_D), _row_spec(_D), _row_spec(1), _row_spec(_D), _row_spec(_D),
        _rep_spec((1, 3)),
        _row_spec(_D),
        _rep_spec((_K + 1, _D, _D)),
        _rep_spec((1, _D)),
    ],
    out_specs=_row_spec(_D),
    out_shape=jax.ShapeDtypeStruct((_N, _D), jnp.float32),
)

_repack_call = pl.pallas_call(
    _repack_body,
    grid=(_EBLKS,),
    in_specs=[pl.BlockSpec((2, _EBLK), lambda i: (0, i))],
    out_specs=[
        pl.BlockSpec((_EROWS, _ECOLS), lambda i: (i, 0)),
        pl.BlockSpec((_EROWS, _ECOLS), lambda i: (i, 0)),
    ],
    out_shape=[
        jax.ShapeDtypeStruct((_EBLKS * _EROWS, _ECOLS), jnp.int32),
        jax.ShapeDtypeStruct((_EBLKS * _EROWS, _ECOLS), jnp.int32),
    ],
)


def _coef_schedule(alphas):
    coef1 = (_PA - _PB) / 2 - (_PA + _PB + 2) / 2 * ((_LO + _HI) / (_HI - _LO))
    coef2 = (_PA + _PB + 2) / (_HI - _LO)
    scheds = [(alphas[0] * coef2, alphas[0] * coef1, alphas[0] * 0.0)]
    for L in range(2, _K + 1):
        coef_l = 2 * L * (L + _PA + _PB) * (2 * L - 2 + _PA + _PB)
        coef_lm1_1 = (2 * L + _PA + _PB - 1) * (2 * L + _PA + _PB) * (2 * L + _PA + _PB - 2)
        coef_lm1_2 = (2 * L + _PA + _PB - 1) * (_PA**2 - _PB**2)
        coef_lm2 = 2 * (L - 1 + _PA) * (L - 1 + _PB) * (2 * L + _PA + _PB)
        tmp1 = alphas[L - 1] * (coef_lm1_1 / coef_l)
        tmp2 = alphas[L - 1] * (coef_lm1_2 / coef_l)
        tmp3 = alphas[L - 1] * alphas[L - 2] * (coef_lm2 / coef_l)
        tmp1_2 = tmp1 * (2.0 / (_HI - _LO))
        tmp2_2 = tmp1 * ((_HI + _LO) / (_HI - _LO)) + tmp2
        scheds.append((tmp1_2, -tmp2_2, -tmp3))
    return scheds


def kernel(x, edge_index, alphas_param, W, bias):
    rowp, colp = _repack_call(edge_index)
    rowp = rowp.reshape(_NW, _NCHT, _CH)
    colp = colp.reshape(_NW, _NCHT, _CH)
    zeros1 = jnp.zeros((_RPT,), jnp.float32)
    ones1 = jnp.ones((_CH,), jnp.float32)
    zeros2 = jnp.zeros((_RPT, _D), jnp.float32)

    deg_kernel, spmm_kernel = _build_sc_kernels()
    degp = deg_kernel(edge_index, zeros1, ones1)
    y, dis = _disy_call(degp, x)

    alphas = jnp.tanh(alphas_param)
    scheds = _coef_schedule(alphas)
    cfs = [jnp.stack(s).reshape(1, 3).astype(jnp.float32) for s in scheds]

    p0, p1 = spmm_kernel(y, colp, rowp, zeros2)
    x1, y = _combine_first_call(p0, p1, dis, x, cfs[0])
    p0, p1 = spmm_kernel(y, colp, rowp, zeros2)
    x2, y = _combine_mid_call(p0, p1, dis, x1, x, cfs[1])
    p0, p1 = spmm_kernel(y, colp, rowp, zeros2)

    wt = W.T.reshape(_K + 1, _D, _D)
    return _combine_mm_call(p0, p1, dis, x2, x1, cfs[2], x, wt, bias.reshape(1, _D))
